# baseline (speedup 1.0000x reference)
import jax
import jax.numpy as jnp
from jax import lax
from jax.experimental import pallas as pl
from jax.experimental.pallas import tpu as pltpu
from jax.experimental.pallas import tpu_sc as plsc

N_NODES = 100_000
N_EDGES = 6_400_000
NPAD = 100_352
CHUNK = 128
NCHUNKS = N_EDGES // CHUNK
BLKC = 40
BLKE = BLKC * CHUNK
NBLOCKS = N_EDGES // BLKE
NW = 32
ZSLICE = NPAD // 16
DSLICE = 6272
_NFULL = NBLOCKS - (NBLOCKS // NW) * NW


def _spmv_body(ei_hbm, mv3_hbm, d_hbm, out_hbm,
               pairb_v, mv_v, dval_v, vals_v, zero_v, d_sp, acc_sh,
               sem_p, sem_m, sem_g, sem_sc):
    c = lax.axis_index("c")
    s = lax.axis_index("s")
    w = s * 2 + c

    QTR = ZSLICE // 4

    @pl.when(s < 15)
    def _():
        for k in range(4):
            doff = pl.multiple_of(s * DSLICE + k * QTR, 8)
            pltpu.sync_copy(d_hbm.at[pl.ds(doff, QTR)], zero_v)
            pltpu.sync_copy(zero_v, d_sp.at[pl.ds(doff, QTR)])

    @pl.when(s == 15)
    def _():
        for k in range(3):
            doff = 15 * DSLICE + k * QTR
            pltpu.sync_copy(d_hbm.at[pl.ds(doff, QTR)], zero_v)
            pltpu.sync_copy(zero_v, d_sp.at[pl.ds(doff, QTR)])
        tail = N_NODES - (15 * DSLICE + 3 * QTR)
        toff = 15 * DSLICE + 3 * QTR
        pltpu.sync_copy(d_hbm.at[pl.ds(toff, tail)],
                        zero_v.at[pl.ds(0, tail)])
        pltpu.sync_copy(zero_v.at[pl.ds(0, tail)],
                        d_sp.at[pl.ds(toff, tail)])

    def _zero(i, _):
        zero_v[pl.ds(i * 16, 16)] = jnp.zeros((16,), jnp.float32)
        return 0
    lax.fori_loop(0, ZSLICE // 4 // 16, _zero, 0)
    for k in range(4):
        offk = pl.multiple_of(s * ZSLICE + k * (ZSLICE // 4), 8)
        pltpu.sync_copy(zero_v, acc_sh.at[pl.ds(offk, ZSLICE // 4)])

    nblk = jnp.where(w < _NFULL, NBLOCKS // NW + 1, NBLOCKS // NW)

    def _in_copies(j, b):
        g = w + NW * j
        chb = pl.multiple_of(g * BLKC, BLKC)
        return (
            pltpu.make_async_copy(ei_hbm.at[pl.ds(chb, BLKC)],
                                  pairb_v.at[b], sem_p.at[b]),
            pltpu.make_async_copy(mv3_hbm.at[pl.ds(chb, BLKC)],
                                  mv_v.at[b], sem_m.at[b]),
        )

    def _issue(j, b):
        for cp in _in_copies(j, b):
            cp.start()

    def _wait_in(j, b):
        for cp in _in_copies(j, b):
            cp.wait()

    def _drain_scatters():
        for ch in range(BLKC):
            pltpu.make_async_copy(mv3_hbm.at[0], vals_v.at[0, ch],
                                  sem_sc).wait()

    def _compute_fire(b, b2):
        gathers = [
            pltpu.async_copy(d_sp.at[pairb_v.at[b, ch, 0]],
                             dval_v.at[ch], sem_g)
            for ch in range(BLKC)
        ]
        ng = CHUNK // 16
        for ch in range(BLKC):
            gathers[ch].wait()
            dv = [dval_v[ch, pl.ds(gg * 16, 16)] for gg in range(ng)]
            mvs = [mv_v[b, ch, pl.ds(gg * 16, 16)] for gg in range(ng)]
            for gg in range(ng):
                vals_v[b2, ch, pl.ds(gg * 16, 16)] = dv[gg] * mvs[gg]
            pltpu.async_copy(vals_v.at[b2, ch],
                             acc_sh.at[pairb_v.at[b, ch, 1]],
                             sem_sc, add=True)

    _issue(0, 0)
    _issue(1, 1)
    plsc.subcore_barrier()

    def _quad(jj, _):
        for b in range(4):
            j = 4 * jj + b

            @pl.when((j >= 2) & (j < nblk))
            def _(j=j):
                _drain_scatters()

            @pl.when(j + 2 < nblk)
            def _(j=j, b=b):
                _issue(j + 2, (b + 2) % 4)

            @pl.when(j < nblk)
            def _(j=j, b=b):
                _wait_in(j, b)
                _compute_fire(b, b % 2)
        return 0
    lax.fori_loop(0, (NBLOCKS // NW + 4) // 4, _quad, 0)

    _drain_scatters()
    _drain_scatters()
    plsc.subcore_barrier()

    @pl.when(s == 0)
    def _():
        nc = pl.multiple_of(c * NPAD, 8)
        pltpu.sync_copy(acc_sh, out_hbm.at[pl.ds(nc, NPAD)])


_spmv = pl.kernel(
    _spmv_body,
    out_type=jax.ShapeDtypeStruct((2 * NPAD,), jnp.float32),
    mesh=plsc.VectorSubcoreMesh(core_axis_name="c", subcore_axis_name="s"),
    compiler_params=pltpu.CompilerParams(needs_layout_passes=False),
    scratch_types=[
        pltpu.VMEM((4, BLKC, 2, CHUNK), jnp.int32),
        pltpu.VMEM((4, BLKC, CHUNK), jnp.float32),
        pltpu.VMEM((BLKC, CHUNK), jnp.float32),
        pltpu.VMEM((2, BLKC, CHUNK), jnp.float32),
        pltpu.VMEM((ZSLICE // 4,), jnp.float32),
        pltpu.VMEM_SHARED((N_NODES,), jnp.float32),
        pltpu.VMEM_SHARED((NPAD,), jnp.float32),
        pltpu.SemaphoreType.DMA((4,)),
        pltpu.SemaphoreType.DMA((4,)),
        pltpu.SemaphoreType.DMA,
        pltpu.SemaphoreType.DMA,
    ],
)


def _finish_body(p_ref, d_ref, r_ref, out_ref):
    ad = p_ref[0] + p_ref[1]
    dd = d_ref[...]
    rr = r_ref[...]
    r_dot_d = jnp.sum(rr * dd)
    d_dot_q = jnp.sum(dd * ad)
    alpha = r_dot_d / (d_dot_q + 1e-6)
    err = alpha * ad - rr
    out_ref[...] = jnp.reshape(jnp.sum(err * err) / N_NODES, (1, 1))


_finish = pl.pallas_call(
    _finish_body,
    out_shape=jax.ShapeDtypeStruct((1, 1), jnp.float32),
)


def kernel(d, residual, edge_index, matrix_values, mask, L_values, batch_vec):
    del mask, L_values, batch_vec
    ei = (edge_index.astype(jnp.int32)
          .reshape(2, NCHUNKS, CHUNK).transpose(1, 0, 2))
    mv3 = matrix_values.reshape(NCHUNKS, CHUNK)
    partials = _spmv(ei, mv3, d)
    pad = NPAD - N_NODES
    d_pad = jnp.pad(d, (0, pad)).reshape(NPAD // 128, 128)
    r_pad = jnp.pad(residual, (0, pad)).reshape(NPAD // 128, 128)
    p = partials.reshape(2, NPAD // 128, 128)
    loss = _finish(p, d_pad, r_pad)
    return loss[0, 0]


# speedup vs baseline: 1.7572x; 1.7572x over previous
import functools

import jax
import jax.numpy as jnp
from jax import lax
from jax.experimental import pallas as pl
from jax.experimental.pallas import tpu as pltpu
from jax.experimental.pallas import tpu_sc as plsc

N_NODES = 100_000
N_EDGES = 6_400_000
NPAD = 100_352
CHUNK = 128
BLKC = 16
BLKE = BLKC * CHUNK
NBLOCKS = N_EDGES // BLKE
NW = 32
ZSLICE = NPAD // 16
_NFULL = NBLOCKS - (NBLOCKS // NW) * NW


def _spmv_body(ei_hbm, mv3_hbm, d_hbm, out_hbm,
               d_v, pairb_v, mv_v, vals_v, zero_v, acc_sh,
               sem_in, sem_sc):
    c = lax.axis_index("c")
    s = lax.axis_index("s")
    w = s * 2 + c

    d_cp = pltpu.make_async_copy(d_hbm, d_v, sem_sc)
    d_cp.start()

    def _zero(i, _):
        zero_v[pl.ds(i * 16, 16)] = jnp.zeros((16,), jnp.float32)
        return 0
    lax.fori_loop(0, ZSLICE // 4 // 16, _zero, 0)
    for k in range(4):
        offk = pl.multiple_of(s * ZSLICE + k * (ZSLICE // 4), 8)
        pltpu.sync_copy(zero_v, acc_sh.at[pl.ds(offk, ZSLICE // 4)])

    nblk = jnp.where(w < _NFULL, NBLOCKS // NW + 1, NBLOCKS // NW)

    def _in_copies(j, b):
        g = w + NW * j
        base = pl.multiple_of(g * BLKE, BLKE)
        chb = pl.multiple_of(g * BLKC, BLKC)
        return (
            pltpu.make_async_copy(ei_hbm.at[pl.ds(chb, BLKC)],
                                  pairb_v.at[b], sem_in.at[b]),
            pltpu.make_async_copy(mv3_hbm.at[pl.ds(chb, BLKC)],
                                  mv_v.at[b], sem_in.at[b]),
        )

    def _issue(j, b):
        for cp in _in_copies(j, b):
            cp.start()

    def _wait_in(j, b):
        for cp in _in_copies(j, b):
            cp.wait()

    def _drain_scatters():
        for ch in range(BLKC):
            pltpu.make_async_copy(mv3_hbm.at[0], vals_v.at[0, ch],
                                  sem_sc).wait()

    def _compute_fire(b, b2):
        for ch in range(BLKC):
            ng = CHUNK // 16
            idxs = [pairb_v[b, ch, 0, pl.ds(gg * 16, 16)] for gg in range(ng)]
            gath = [plsc.load_gather(d_v, [ix]) for ix in idxs]
            mvs = [mv_v[b, ch, pl.ds(gg * 16, 16)] for gg in range(ng)]
            for gg in range(ng):
                vals_v[b2, ch, pl.ds(gg * 16, 16)] = gath[gg] * mvs[gg]
            pltpu.async_copy(vals_v.at[b2, ch],
                             acc_sh.at[pairb_v.at[b, ch, 1]],
                             sem_sc, add=True)

    _issue(0, 0)
    _issue(1, 1)
    d_cp.wait()
    plsc.subcore_barrier()

    def _six(jj, _):
        for bb in range(6):
            j = 6 * jj + bb
            b, b2 = bb % 3, bb % 2

            @pl.when((j >= 1) & (j < nblk))
            def _(j=j):
                _drain_scatters()

            @pl.when(j + 2 < nblk)
            def _(j=j, b=b):
                _issue(j + 2, (b + 2) % 3)

            @pl.when(j < nblk)
            def _(j=j, b=b, b2=b2):
                _wait_in(j, b)
                _compute_fire(b, b2)
        return 0
    lax.fori_loop(0, (NBLOCKS // NW + 6) // 6, _six, 0)

    _drain_scatters()
    plsc.subcore_barrier()

    @pl.when(s == 0)
    def _():
        nc = pl.multiple_of(c * NPAD, 8)
        pltpu.sync_copy(acc_sh, out_hbm.at[pl.ds(nc, NPAD)])


_spmv = pl.kernel(
    _spmv_body,
    out_type=jax.ShapeDtypeStruct((2 * NPAD,), jnp.float32),
    mesh=plsc.VectorSubcoreMesh(core_axis_name="c", subcore_axis_name="s"),
    compiler_params=pltpu.CompilerParams(needs_layout_passes=False),
    scratch_types=[
        pltpu.VMEM((N_NODES,), jnp.float32),
        pltpu.VMEM((3, BLKC, 2, CHUNK), jnp.int32),
        pltpu.VMEM((3, BLKC, CHUNK), jnp.float32),
        pltpu.VMEM((2, BLKC, CHUNK), jnp.float32),
        pltpu.VMEM((ZSLICE // 4,), jnp.float32),
        pltpu.VMEM_SHARED((NPAD,), jnp.float32),
        pltpu.SemaphoreType.DMA((3,)),
        pltpu.SemaphoreType.DMA,
    ],
)


def _finish_body(p_ref, d_ref, r_ref, out_ref):
    ad = p_ref[0] + p_ref[1]
    dd = d_ref[...]
    rr = r_ref[...]
    r_dot_d = jnp.sum(rr * dd)
    d_dot_q = jnp.sum(dd * ad)
    alpha = r_dot_d / (d_dot_q + 1e-6)
    err = alpha * ad - rr
    out_ref[...] = jnp.reshape(jnp.sum(err * err) / N_NODES, (1, 1))


_finish = pl.pallas_call(
    _finish_body,
    out_shape=jax.ShapeDtypeStruct((1, 1), jnp.float32),
)


def kernel(d, residual, edge_index, matrix_values, mask, L_values, batch_vec):
    del mask, L_values, batch_vec
    ei = (edge_index.astype(jnp.int32)
          .reshape(2, NBLOCKS * BLKC, CHUNK).transpose(1, 0, 2))
    mv3 = matrix_values.reshape(NBLOCKS * BLKC, CHUNK)
    partials = _spmv(ei, mv3, d)
    pad = NPAD - N_NODES
    d_pad = jnp.pad(d, (0, pad)).reshape(NPAD // 128, 128)
    r_pad = jnp.pad(residual, (0, pad)).reshape(NPAD // 128, 128)
    p = partials.reshape(2, NPAD // 128, 128)
    loss = _finish(p, d_pad, r_pad)
    return loss[0, 0]


# speedup vs baseline: 1.8010x; 1.0249x over previous
import jax
import jax.numpy as jnp
from jax import lax
from jax.experimental import pallas as pl
from jax.experimental.pallas import tpu as pltpu
from jax.experimental.pallas import tpu_sc as plsc

N_NODES = 100_000
N_EDGES = 6_400_000
NPAD = 100_352
CHUNK = 128
BLKC = 16
BLKE = BLKC * CHUNK
NBLOCKS = N_EDGES // BLKE
NW = 32
ZSLICE = NPAD // 16
_NFULL = NBLOCKS - (NBLOCKS // NW) * NW


def _spmv_body(ei_hbm, mv3_hbm, d_hbm, out_hbm,
               d_v, pairb_v, mv_v, vals_v, zero_v, acc_sh,
               sem_in, sem_sc):
    c = lax.axis_index("c")
    s = lax.axis_index("s")
    w = s * 2 + c

    d_cp = pltpu.make_async_copy(d_hbm, d_v, sem_sc)
    d_cp.start()

    def _zero(i, _):
        zero_v[pl.ds(i * 16, 16)] = jnp.zeros((16,), jnp.float32)
        return 0
    lax.fori_loop(0, ZSLICE // 2 // 16, _zero, 0)
    off = pl.multiple_of(s * ZSLICE, 8)
    pltpu.sync_copy(zero_v, acc_sh.at[pl.ds(off, ZSLICE // 2)])
    off2 = pl.multiple_of(s * ZSLICE + ZSLICE // 2, 8)
    pltpu.sync_copy(zero_v, acc_sh.at[pl.ds(off2, ZSLICE // 2)])

    nblk = jnp.where(w < _NFULL, NBLOCKS // NW + 1, NBLOCKS // NW)

    def _in_copies(j, b):
        g = w + NW * j
        base = pl.multiple_of(g * BLKE, BLKE)
        chb = pl.multiple_of(g * BLKC, BLKC)
        return (
            pltpu.make_async_copy(ei_hbm.at[pl.ds(chb, BLKC)],
                                  pairb_v.at[b], sem_in.at[b]),
            pltpu.make_async_copy(mv3_hbm.at[pl.ds(chb, BLKC)],
                                  mv_v.at[b], sem_in.at[b]),
        )

    def _issue(j, b):
        for cp in _in_copies(j, b):
            cp.start()

    def _wait_in(j, b):
        for cp in _in_copies(j, b):
            cp.wait()

    def _compute_scatter(b):
        handles = []
        for ch in range(BLKC):
            base = ch * CHUNK
            ng = CHUNK // 16
            idxs = [pairb_v[b, ch, 0, pl.ds(gg * 16, 16)] for gg in range(ng)]
            gath = [plsc.load_gather(d_v, [ix]) for ix in idxs]
            mvs = [mv_v[b, ch, pl.ds(gg * 16, 16)] for gg in range(ng)]
            for gg in range(ng):
                vals_v[ch, pl.ds(gg * 16, 16)] = gath[gg] * mvs[gg]
            handles.append(pltpu.async_copy(
                vals_v.at[ch],
                acc_sh.at[pairb_v.at[b, ch, 1]], sem_sc, add=True))
        for h in handles:
            h.wait()

    _issue(0, 0)
    _issue(1, 1)
    d_cp.wait()
    plsc.subcore_barrier()

    def _trip(jj, _):
        for b in (0, 1, 2):
            j = 3 * jj + b

            @pl.when(j + 2 < nblk)
            def _(j=j, b=b):
                _issue(j + 2, (b + 2) % 3)

            @pl.when(j < nblk)
            def _(j=j, b=b):
                _wait_in(j, b)
                _compute_scatter(b)
        return 0
    lax.fori_loop(0, (NBLOCKS // NW + 3) // 3, _trip, 0)

    plsc.subcore_barrier()

    @pl.when(s == 0)
    def _():
        nc = pl.multiple_of(c * NPAD, 8)
        pltpu.sync_copy(acc_sh, out_hbm.at[pl.ds(nc, NPAD)])


_spmv = pl.kernel(
    _spmv_body,
    out_type=jax.ShapeDtypeStruct((2 * NPAD,), jnp.float32),
    mesh=plsc.VectorSubcoreMesh(core_axis_name="c", subcore_axis_name="s"),
    compiler_params=pltpu.CompilerParams(needs_layout_passes=False),
    scratch_types=[
        pltpu.VMEM((N_NODES,), jnp.float32),
        pltpu.VMEM((3, BLKC, 2, CHUNK), jnp.int32),
        pltpu.VMEM((3, BLKC, CHUNK), jnp.float32),
        pltpu.VMEM((BLKC, CHUNK), jnp.float32),
        pltpu.VMEM((ZSLICE // 2,), jnp.float32),
        pltpu.VMEM_SHARED((NPAD,), jnp.float32),
        pltpu.SemaphoreType.DMA((3,)),
        pltpu.SemaphoreType.DMA,
    ],
)


def _finish_body(p_ref, d_ref, r_ref, out_ref):
    ad = p_ref[0] + p_ref[1]
    dd = d_ref[...]
    rr = r_ref[...]
    r_dot_d = jnp.sum(rr * dd)
    d_dot_q = jnp.sum(dd * ad)
    alpha = r_dot_d / (d_dot_q + 1e-6)
    err = alpha * ad - rr
    out_ref[...] = jnp.reshape(jnp.sum(err * err) / N_NODES, (1, 1))


_finish = pl.pallas_call(
    _finish_body,
    out_shape=jax.ShapeDtypeStruct((1, 1), jnp.float32),
)


def kernel(d, residual, edge_index, matrix_values, mask, L_values, batch_vec):
    del mask, L_values, batch_vec
    ei = (edge_index.astype(jnp.int32)
          .reshape(2, NBLOCKS * BLKC, CHUNK).transpose(1, 0, 2))
    mv3 = matrix_values.reshape(NBLOCKS * BLKC, CHUNK)
    partials = _spmv(ei, mv3, d)
    pad = NPAD - N_NODES
    d_pad = jnp.pad(d, (0, pad)).reshape(NPAD // 128, 128)
    r_pad = jnp.pad(residual, (0, pad)).reshape(NPAD // 128, 128)
    p = partials.reshape(2, NPAD // 128, 128)
    loss = _finish(p, d_pad, r_pad)
    return loss[0, 0]
